# baseline (speedup 1.0000x reference)
import functools

import jax
import jax.numpy as jnp
from jax import lax
from jax.experimental import pallas as pl
from jax.experimental.pallas import tpu as pltpu
from jax.experimental.pallas import tpu_sc as plsc

C = 16
K = 10
ALPHA = 0.9
BP = 256
G = 8
NC = 2
NS = 16


def _gather_rows(a, idxp):
    n = a.shape[1]
    npad = idxp.shape[0]
    nw = NC * NS
    s = npad // nw
    mesh = plsc.VectorSubcoreMesh(core_axis_name="c", subcore_axis_name="s")

    @functools.partial(
        pl.kernel, mesh=mesh,
        out_type=jax.ShapeDtypeStruct((npad, n), jnp.float32),
        compiler_params=pltpu.CompilerParams(use_tc_tiling_on_sc=False),
        scratch_types=[
            pltpu.VMEM((G,), jnp.int32),
            pltpu.VMEM((G, n), jnp.float32),
            pltpu.SemaphoreType.DMA,
        ],
    )
    def sc_gather(idx_hbm, a_hbm, out_hbm, idx_v, rows_v, sem):
        wid = lax.axis_index("s") * NC + lax.axis_index("c")
        start = wid * s

        def chunk(ci, carry):
            base = pl.multiple_of(start + ci * G, G)
            pltpu.sync_copy(idx_hbm.at[pl.ds(base, G)], idx_v)
            pltpu.async_copy(a_hbm.at[idx_v], rows_v, sem).wait()
            pltpu.sync_copy(rows_v, out_hbm.at[pl.ds(base, G)])
            return carry

        lax.fori_loop(0, s // G, chunk, 0)

    return sc_gather(idxp, a)


def _lp_kernel(f_ref, ranks_ref, af_hbm, cb_ref,
               out_ref, a16_hbm, abf_ref, ab16_ref, u_ref, src_ref,
               semf_ref, sem16_ref, semwb_ref, *, be, n, w):
    k = pl.program_id(0)
    f = f_ref[0]
    nb = (f + BP - 1) // BP

    @pl.when(k == 0)
    def _():
        u_ref[...] = jnp.zeros_like(u_ref)

    def copy_f32(r, slot):
        return pltpu.make_async_copy(
            af_hbm.at[pl.ds(r * BP, BP), :], abf_ref.at[slot],
            semf_ref.at[slot])

    def copy_b16_in(r, slot):
        return pltpu.make_async_copy(
            a16_hbm.at[pl.ds(r * BP, BP), :], ab16_ref.at[slot],
            sem16_ref.at[slot])

    def copy_b16_out(r, slot):
        return pltpu.make_async_copy(
            ab16_ref.at[slot], a16_hbm.at[pl.ds(r * BP, BP), :],
            semwb_ref.at[slot])

    @pl.when(nb > 0)
    def _():
        @pl.when(k == 0)
        def _():
            copy_f32(0, 0).start()

        @pl.when((k > 0) & (k < K))
        def _():
            copy_b16_in(0, 0).start()

    def expand(i, carry):
        sl = pl.ds(i * be, be)
        base = ranks_ref[i * be]
        base8 = (base // 8) * 8
        cb = cb_ref[sl, :]
        relf = cb[:, 32:33] - base8.astype(jnp.float32)
        iota = lax.broadcasted_iota(
            jnp.int32, (1, w), 1).astype(jnp.float32)
        e = (relf == iota).astype(jnp.float32)
        uw = u_ref[pl.ds(base8, w), :]
        ex = jnp.dot(e, uw, preferred_element_type=jnp.float32)
        val = cb[:, 0:16] + cb[:, 16:32] * ex
        src_ref[sl, :] = val.astype(jnp.bfloat16)

        @pl.when(k == K)
        def _():
            out_ref[sl, :] = val
        return carry

    lax.fori_loop(0, n // be, expand, 0)

    @pl.when(k == 0)
    def _():
        def mm0(r, carry):
            slot = lax.rem(r, 2)

            @pl.when(r + 1 < nb)
            def _():
                copy_f32(r + 1, 1 - slot).start()

            @pl.when(r >= 2)
            def _():
                copy_b16_out(r - 2, slot).wait()

            copy_f32(r, slot).wait()
            a16 = abf_ref[slot].astype(jnp.bfloat16)
            ab16_ref[slot] = a16
            copy_b16_out(r, slot).start()
            z = jnp.dot(a16, src_ref[...],
                        preferred_element_type=jnp.float32)
            usl = pl.ds(r * BP, BP)
            u_ref[usl, :] = jnp.clip(
                ALPHA * z + (1.0 - ALPHA) * u_ref[usl, :], 0.0, 1.0)
            return carry

        lax.fori_loop(0, nb, mm0, 0)

        @pl.when(nb > 1)
        def _():
            copy_b16_out(nb - 2, lax.rem(nb - 2, 2)).wait()

        @pl.when(nb > 0)
        def _():
            copy_b16_out(nb - 1, lax.rem(nb - 1, 2)).wait()

    @pl.when((k > 0) & (k < K))
    def _():
        def mm(r, carry):
            slot = lax.rem(r, 2)

            @pl.when(r + 1 < nb)
            def _():
                copy_b16_in(r + 1, 1 - slot).start()

            copy_b16_in(r, slot).wait()
            z = jnp.dot(ab16_ref[slot], src_ref[...],
                        preferred_element_type=jnp.float32)
            usl = pl.ds(r * BP, BP)
            u_ref[usl, :] = jnp.clip(
                ALPHA * z + (1.0 - ALPHA) * u_ref[usl, :], 0.0, 1.0)
            return carry

        lax.fori_loop(0, nb, mm, 0)


def kernel(homo_adj, y, train_mask):
    n = homo_adj.shape[0]
    nw = NC * NS
    npad = -(-n // (G * nw)) * (G * nw)
    be = 400 if n % 400 == 0 else max(
        d for d in (200, 40, 16, 8) if n % d == 0)
    w = be + 8

    free = jnp.logical_not(train_mask)
    perm = jnp.argsort(jnp.where(free, 0, 1), stable=True)
    freei = free.astype(jnp.int32)
    rank = jnp.cumsum(freei) - freei
    f = jnp.sum(freei)

    idxp = jnp.where(jnp.arange(npad, dtype=jnp.int32) < f,
                     jnp.pad(perm.astype(jnp.int32), (0, npad - n)), 0)
    afp = _gather_rows(homo_adj, idxp)

    y_oh = jax.nn.one_hot(y.astype(jnp.int32), C, dtype=jnp.float32)
    maskf = jnp.broadcast_to(
        train_mask.astype(jnp.float32)[:, None], (n, C))
    combo = jnp.concatenate(
        [maskf * y_oh, 1.0 - maskf,
         jnp.broadcast_to(rank.astype(jnp.float32)[:, None], (n, C))],
        axis=1)

    body = functools.partial(_lp_kernel, be=be, n=n, w=w)
    out, _ = pl.pallas_call(
        body,
        grid=(K + 1,),
        in_specs=[
            pl.BlockSpec(memory_space=pltpu.SMEM),
            pl.BlockSpec(memory_space=pltpu.SMEM),
            pl.BlockSpec(memory_space=pl.ANY),
            pl.BlockSpec((n, 3 * C), lambda k: (0, 0)),
        ],
        out_specs=[
            pl.BlockSpec((n, C), lambda k: (0, 0)),
            pl.BlockSpec(memory_space=pl.ANY),
        ],
        out_shape=[
            jax.ShapeDtypeStruct((n, C), jnp.float32),
            jax.ShapeDtypeStruct((npad, n), jnp.bfloat16),
        ],
        scratch_shapes=[
            pltpu.VMEM((2, BP, n), jnp.float32),
            pltpu.VMEM((2, BP, n), jnp.bfloat16),
            pltpu.VMEM((npad, C), jnp.float32),
            pltpu.VMEM((n, C), jnp.bfloat16),
            pltpu.SemaphoreType.DMA((2,)),
            pltpu.SemaphoreType.DMA((2,)),
            pltpu.SemaphoreType.DMA((2,)),
        ],
    )(f[None], rank, afp, combo)
    return out


# speedup vs baseline: 2.4395x; 2.4395x over previous
import functools

import jax
import jax.numpy as jnp
from jax.experimental import pallas as pl
from jax.experimental.pallas import tpu as pltpu

C = 16
K = 10
ALPHA = 0.9


def _step1_kernel(a_ref, yoh_ref, m_ref, a16_ref, out1_ref, *, bi, n):
    i = pl.program_id(0)
    base = m_ref[...] * yoh_ref[...]
    z = jnp.dot(a_ref[...], base[:n, :], preferred_element_type=jnp.float32)

    sl = pl.ds(i * bi, bi)
    m_i = m_ref[sl, :]
    yoh_i = yoh_ref[sl, :]
    val = jnp.clip(ALPHA * z + (1.0 - ALPHA) * (m_i * yoh_i), 0.0, 1.0)
    out1_ref[...] = jnp.where(m_i > 0.0, yoh_i, val)
    a16_ref[...] = a_ref[...].astype(jnp.bfloat16)


def _steps_kernel(a_ref, yoh_ref, m_ref, o1_ref, out_ref, buf_ref,
                  *, bi, ni, n, ks):
    k = pl.program_id(0)
    i = pl.program_id(1)
    cur = jax.lax.rem(k, 2)

    src = jnp.where(k == 0, o1_ref[:n, :],
                    buf_ref[cur, :n, :]).astype(jnp.bfloat16)
    z = jnp.dot(a_ref[...], src, preferred_element_type=jnp.float32)

    sl = pl.ds(i * bi, bi)
    m_i = m_ref[sl, :]
    yoh_i = yoh_ref[sl, :]
    old_i = jnp.where(k == 0, o1_ref[sl, :], buf_ref[cur, sl, :])
    val = jnp.clip(ALPHA * z + (1.0 - ALPHA) * old_i, 0.0, 1.0)
    new = jnp.where(m_i > 0.0, yoh_i, val)
    buf_ref[1 - cur, sl, :] = new

    @pl.when(k == ks - 1)
    def _():
        out_ref[...] = new


def kernel(homo_adj, y, train_mask):
    n = homo_adj.shape[0]
    b1 = 400
    b2 = 800
    n1 = -(-n // b1)
    n2 = -(-n // b2)
    np1 = n1 * b1
    np2 = n2 * b2
    npx = max(np1, np2)

    y_oh = jnp.pad(jax.nn.one_hot(y.astype(jnp.int32), C, dtype=jnp.float32),
                   ((0, npx - n), (0, 0)))
    maskf = jnp.pad(jnp.broadcast_to(
        train_mask.astype(jnp.float32)[:, None], (n, C)),
        ((0, npx - n), (0, 0)))

    a16, out1 = pl.pallas_call(
        functools.partial(_step1_kernel, bi=b1, n=n),
        grid=(n1,),
        in_specs=[
            pl.BlockSpec((b1, n), lambda i: (i, 0)),
            pl.BlockSpec((np1, C), lambda i: (0, 0)),
            pl.BlockSpec((np1, C), lambda i: (0, 0)),
        ],
        out_specs=[
            pl.BlockSpec((b1, n), lambda i: (i, 0)),
            pl.BlockSpec((b1, C), lambda i: (i, 0)),
        ],
        out_shape=[
            jax.ShapeDtypeStruct((np1, n), jnp.bfloat16),
            jax.ShapeDtypeStruct((np1, C), jnp.float32),
        ],
    )(homo_adj, y_oh[:np1], maskf[:np1])

    ks = K - 1
    out = pl.pallas_call(
        functools.partial(_steps_kernel, bi=b2, ni=n2, n=n, ks=ks),
        grid=(ks, n2),
        in_specs=[
            pl.BlockSpec((b2, n), lambda k, i: (i, 0)),
            pl.BlockSpec((np2, C), lambda k, i: (0, 0)),
            pl.BlockSpec((np2, C), lambda k, i: (0, 0)),
            pl.BlockSpec((np2, C), lambda k, i: (0, 0)),
        ],
        out_specs=pl.BlockSpec(
            (b2, C),
            lambda k, i: (jax.lax.select(k == ks - 1, i, n2), 0)),
        out_shape=jax.ShapeDtypeStruct(((n2 + 1) * b2, C), jnp.float32),
        scratch_shapes=[
            pltpu.VMEM((2, np2, C), jnp.float32),
        ],
    )(a16, y_oh[:np2], maskf[:np2],
      out1[:np2] if np1 >= np2 else jnp.pad(out1, ((0, np2 - np1), (0, 0))))
    return out[:n]
